# Initial kernel scaffold; baseline (speedup 1.0000x reference)
import jax
import jax.numpy as jnp
from jax.experimental import pallas as pl


def kernel(input_batch, Wr, br, W1, b1, g1, be1, W2, b2, g2, be2):
    raise NotImplementedError("write your pallas kernel here")


# speedup vs baseline: 2.8883x; 2.8883x over previous
import functools

import jax
import jax.numpy as jnp
from jax import lax
from jax.experimental import pallas as pl
from jax.experimental.pallas import tpu as pltpu
from jax.experimental.pallas import tpu_sc as plsc

_E = 8
_D = 1024
_TM = 128


def _router_body(x_ref, wr_ref, br_ref, pos_ref, off_ref):
    x = x_ref[...]
    t = x.shape[0]
    logits = jnp.dot(x, wr_ref[...], preferred_element_type=jnp.float32)
    logits = logits + br_ref[...]

    best = logits[:, 0:1]
    besti = jnp.zeros((t, 1), jnp.int32)
    for e in range(1, _E):
        c = logits[:, e : e + 1] > best
        best = jnp.where(c, logits[:, e : e + 1], best)
        besti = jnp.where(c, e, besti)

    eids = lax.broadcasted_iota(jnp.int32, (t, _E), 1)
    a = (besti == eids).astype(jnp.float32)

    r = lax.broadcasted_iota(jnp.int32, (t, t), 0)
    c = lax.broadcasted_iota(jnp.int32, (t, t), 1)
    tri = (r > c).astype(jnp.bfloat16)
    cum = jnp.dot(tri, a.astype(jnp.bfloat16),
                  preferred_element_type=jnp.float32)

    counts = jnp.sum(a, axis=0, keepdims=True)
    offs = [jnp.zeros((1, 1), jnp.float32)]
    for e in range(1, _E):
        offs.append(offs[-1] + counts[:, e - 1 : e])
    off = jnp.concatenate(offs, axis=1)

    pos = jnp.sum(a * (off + cum), axis=1, keepdims=True)
    pos_ref[...] = pos.astype(jnp.int32)

    off_pad = jnp.concatenate(
        [off, off[:, _E - 1 :] + counts[:, _E - 1 :]]
        + [jnp.full((1, 1), t, jnp.float32)] * (16 - _E - 1),
        axis=1,
    )
    off_ref[...] = off_pad.astype(jnp.int32)


def _run_router(x):
    t = x.shape[0]
    return pl.pallas_call(
        _router_body,
        out_shape=(
            jax.ShapeDtypeStruct((t, 1), jnp.int32),
            jax.ShapeDtypeStruct((1, 16), jnp.int32),
        ),
    )


def _expert_body(off_ref, x_ref, w_ref, b_ref, g_ref, be_ref, out_ref):
    i = pl.program_id(0)
    s0 = i * _TM
    x = x_ref[...]

    lo = jnp.int32(0)
    hi = jnp.int32(0)
    for e in range(1, _E):
        lo += jnp.where(off_ref[e] <= s0, 1, 0).astype(jnp.int32)
        hi += jnp.where(off_ref[e] <= s0 + _TM - 1, 1, 0).astype(jnp.int32)

    srow = s0 + lax.broadcasted_iota(jnp.int32, (_TM, 1), 0)

    def body(e, acc):
        y = jnp.dot(x, w_ref[e], preferred_element_type=jnp.float32)
        y = jax.nn.relu(y + b_ref[e][None, :])
        mu = jnp.mean(y, axis=-1, keepdims=True)
        var = jnp.mean((y - mu) ** 2, axis=-1, keepdims=True)
        y = (y - mu) / jnp.sqrt(var + 1e-5) * g_ref[e][None, :] + be_ref[e][None, :]
        seg_lo = off_ref[e]
        seg_hi = off_ref[e + 1]
        mask = (srow >= seg_lo) & (srow < seg_hi)
        return acc + jnp.where(mask, y, 0.0)

    acc = lax.fori_loop(lo, hi + 1, body, jnp.zeros((_TM, _D), jnp.float32))
    out_ref[...] = acc


def _run_expert_layer(t):
    grid = t // _TM
    return pl.pallas_call(
        _expert_body,
        grid_spec=pltpu.PrefetchScalarGridSpec(
            num_scalar_prefetch=1,
            grid=(grid,),
            in_specs=[
                pl.BlockSpec((_TM, _D), lambda i, off: (i, 0)),
                pl.BlockSpec((_E, _D, _D), lambda i, off: (0, 0, 0)),
                pl.BlockSpec((_E, _D), lambda i, off: (0, 0)),
                pl.BlockSpec((_E, _D), lambda i, off: (0, 0)),
                pl.BlockSpec((_E, _D), lambda i, off: (0, 0)),
            ],
            out_specs=pl.BlockSpec((_TM, _D), lambda i, off: (i, 0)),
        ),
        out_shape=jax.ShapeDtypeStruct((t, _D), jnp.float32),
        compiler_params=pltpu.CompilerParams(
            dimension_semantics=("arbitrary",),
        ),
    )


def _sc_permute(t, direction):
    info = plsc.get_sparse_core_info()
    nw = info.num_cores * info.num_subcores
    rows_per_w = t // nw
    mesh = plsc.VectorSubcoreMesh(core_axis_name="c", subcore_axis_name="s")

    def body(src_hbm, idx_hbm, out_hbm, idx_v, rows_v, sem):
        wid = lax.axis_index("s") * info.num_cores + lax.axis_index("c")
        base = wid * rows_per_w
        pltpu.sync_copy(idx_hbm.at[pl.ds(base, rows_per_w)], idx_v)
        if direction == "gather":
            pltpu.async_copy(src_hbm.at[idx_v], rows_v, sem).wait()
            pltpu.sync_copy(rows_v, out_hbm.at[pl.ds(base, rows_per_w)])
        else:
            pltpu.sync_copy(src_hbm.at[pl.ds(base, rows_per_w)], rows_v)
            pltpu.async_copy(rows_v, out_hbm.at[idx_v], sem).wait()

    return pl.kernel(
        body,
        out_type=jax.ShapeDtypeStruct((t, _D), jnp.float32),
        mesh=mesh,
        scratch_types=[
            pltpu.VMEM((rows_per_w,), jnp.int32),
            pltpu.VMEM((rows_per_w, _D), jnp.float32),
            pltpu.SemaphoreType.DMA,
        ],
    )


def kernel(input_batch, Wr, br, W1, b1, g1, be1, W2, b2, g2, be2):
    b, s, d = input_batch.shape
    t = b * s
    x = input_batch.reshape(t, d)

    pos2d, off2d = _run_router(x)(x, Wr, br.reshape(1, _E))
    pos = pos2d.reshape(t)
    offsets = off2d.reshape(16)

    x_sorted = _sc_permute(t, "scatter")(x, pos)
    h1 = _run_expert_layer(t)(offsets, x_sorted, W1, b1, g1, be1)
    h2 = _run_expert_layer(t)(offsets, h1, W2, b2, g2, be2)
    out = _sc_permute(t, "gather")(h2, pos)
    return out.reshape(b, s, d)
